# baseline (speedup 1.0000x reference)
import jax
import jax.numpy as jnp
from jax.experimental import pallas as pl

U = 10
M = 1000
N = U * M
E_EDGES = 160000
H = 8
C = 64
D = H * C
EMB = 64
AH = 64
HID = 128
NUM_LAYERS = 8


def kernel(mission_coords, edge_index, batch, uavs_info, action_mask, speeds, dist_matrix, timetogo_matrix, params):
    src = edge_index[0]
    dst = edge_index[1]
    mask_e = action_mask[..., None]
    sp_e = jnp.broadcast_to(speeds[:, None, None], (U, M, 1))
    d_e = dist_matrix[..., None]
    t_e = timetogo_matrix[..., None]
    mc_e = jnp.broadcast_to(mission_coords[None, :, :], (U, M, 2))
    x = jnp.concatenate([mc_e, mask_e, sp_e, d_e, t_e], axis=-1).reshape(N, 6)
    for l in range(NUM_LAYERS):
        if f"l{l}_Wres" in params:
            residual = x @ params[f"l{l}_Wres"] + params[f"l{l}_Wres_b"]
        else:
            residual = x
        q = (x @ params[f"l{l}_Wq"] + params[f"l{l}_Wq_b"]).reshape(N, H, C)
        k = (x @ params[f"l{l}_Wk"] + params[f"l{l}_Wk_b"]).reshape(N, H, C)
        v = (x @ params[f"l{l}_Wv"] + params[f"l{l}_Wv_b"]).reshape(N, H, C)
        alpha = (q[dst] * k[src]).sum(-1) / jnp.sqrt(float(C))
        amax = jax.ops.segment_max(alpha, dst, num_segments=N)
        amax = jnp.where(jnp.isfinite(amax), amax, 0.0)
        ex = jnp.exp(alpha - amax[dst])
        den = jax.ops.segment_sum(ex, dst, num_segments=N)
        a = ex / (den[dst] + 1e-16)
        msg = v[src] * a[:, :, None]
        agg = jax.ops.segment_sum(msg, dst, num_segments=N).reshape(N, D)
        skip = x @ params[f"l{l}_Wskip"] + params[f"l{l}_Wskip_b"]
        x = jax.nn.relu(agg + skip) + residual
    x = x @ params["Wout"] + params["Wout_b"]
    uav_emb = x.reshape(U, M, EMB).sum(axis=1)
    comb = jnp.concatenate([uavs_info, uav_emb, speeds[:, None]], axis=-1)

    def head(pre):
        p = comb @ params[f"{pre}_proj_W"] + params[f"{pre}_proj_b"]
        dh = AH // H
        qh = (p @ params[f"{pre}_attn_Wq"] + params[f"{pre}_attn_bq"]).reshape(U, H, dh)
        kh = (p @ params[f"{pre}_attn_Wk"] + params[f"{pre}_attn_bk"]).reshape(U, H, dh)
        vh = (p @ params[f"{pre}_attn_Wv"] + params[f"{pre}_attn_bv"]).reshape(U, H, dh)
        att = jnp.einsum('qhd,khd->hqk', qh, kh) / jnp.sqrt(float(dh))
        att = jax.nn.softmax(att, axis=-1)
        o = jnp.einsum('hqk,khd->qhd', att, vh).reshape(U, AH)
        o = o @ params[f"{pre}_attn_Wo"] + params[f"{pre}_attn_bo"]
        feat = jax.nn.relu(o @ params[f"{pre}_sh_W"] + params[f"{pre}_sh_b"])
        return feat

    action_logits = head("actor") @ params["actor_out_W"] + params["actor_out_b"]
    state_values = (head("critic") @ params["critic_out_W"] + params["critic_out_b"]).squeeze()
    return action_logits, state_values


# speedup vs baseline: 9.8627x; 9.8627x over previous
import functools
import math

import jax
import jax.numpy as jnp
from jax import lax
from jax.experimental import pallas as pl
from jax.experimental.pallas import tpu as pltpu
from jax.experimental.pallas import tpu_sc as plsc

U = 10
M = 1000
N = U * M
E = 160000
H = 8
C = 64
D = H * C
EMB = 64
AH = 64
HID = 128
NUM_LAYERS = 8

NPAD = 10240
NPW = 320
NWORK = 32
EB = 64
EPAD = E + 256

F32 = jnp.float32


def _proj0_body(x_ref, w_ref, b_ref, q_ref, k_ref, v_ref, s_ref, r_ref):
  y = jnp.dot(x_ref[...], w_ref[...], preferred_element_type=F32) + b_ref[0]
  q_ref[...] = y[:, 0:512]
  k_ref[...] = y[:, 512:1024]
  v_ref[...] = y[:, 1024:1536]
  s_ref[...] = y[:, 1536:2048]
  r_ref[...] = y[:, 2048:2560]


def _proj0(x0p, w, b):
  bm = 512
  outs = [jax.ShapeDtypeStruct((NPAD, 512), F32)] * 5
  return pl.pallas_call(
      _proj0_body,
      grid=(NPAD // bm,),
      in_specs=[
          pl.BlockSpec((bm, 128), lambda i: (i, 0)),
          pl.BlockSpec((128, 2560), lambda i: (0, 0)),
          pl.BlockSpec((1, 2560), lambda i: (0, 0)),
      ],
      out_specs=[pl.BlockSpec((bm, 512), lambda i: (i, 0))] * 5,
      out_shape=outs,
  )(x0p, w, b)


def _projc_body(agg_ref, sk_ref, res_ref, w_ref, b_ref,
                x_ref, q_ref, k_ref, v_ref, s_ref):
  x = jnp.maximum(agg_ref[...] + sk_ref[...], 0.0) + res_ref[...]
  x_ref[...] = x
  y = jnp.dot(x, w_ref[...], preferred_element_type=F32) + b_ref[0]
  q_ref[...] = y[:, 0:512]
  k_ref[...] = y[:, 512:1024]
  v_ref[...] = y[:, 1024:1536]
  s_ref[...] = y[:, 1536:2048]


def _projc(agg, sk, res, w, b):
  bm = 256
  outs = [jax.ShapeDtypeStruct((NPAD, 512), F32)] * 5
  return pl.pallas_call(
      _projc_body,
      grid=(NPAD // bm,),
      in_specs=[
          pl.BlockSpec((bm, 512), lambda i: (i, 0)),
          pl.BlockSpec((bm, 512), lambda i: (i, 0)),
          pl.BlockSpec((bm, 512), lambda i: (i, 0)),
          pl.BlockSpec((512, 2048), lambda i: (0, 0)),
          pl.BlockSpec((1, 2048), lambda i: (0, 0)),
      ],
      out_specs=[pl.BlockSpec((bm, 512), lambda i: (i, 0))] * 5,
      out_shape=outs,
  )(agg, sk, res, w, b)


def _emb_body(agg_ref, sk_ref, res_ref, w_ref, b_ref, out_ref):
  u = pl.program_id(0)
  j = pl.program_id(1)

  @pl.when(jnp.logical_and(u == 0, j == 0))
  def _():
    out_ref[...] = jnp.zeros_like(out_ref)

  x = jnp.maximum(agg_ref[...] + sk_ref[...], 0.0) + res_ref[...]
  o = jnp.dot(x, w_ref[...], preferred_element_type=F32)
  row = jnp.sum(o, axis=0, keepdims=True) + 200.0 * b_ref[0:1, :]
  out_ref[pl.ds(u, 1), :] = out_ref[pl.ds(u, 1), :] + row


def _embred(agg, sk, res, w, b):
  bm = 200
  return pl.pallas_call(
      _emb_body,
      grid=(U, 5),
      in_specs=[
          pl.BlockSpec((bm, 512), lambda u, j: (u * 5 + j, 0)),
          pl.BlockSpec((bm, 512), lambda u, j: (u * 5 + j, 0)),
          pl.BlockSpec((bm, 512), lambda u, j: (u * 5 + j, 0)),
          pl.BlockSpec((512, 64), lambda u, j: (0, 0)),
          pl.BlockSpec((1, 64), lambda u, j: (0, 0)),
      ],
      out_specs=pl.BlockSpec((16, 64), lambda u, j: (0, 0)),
      out_shape=jax.ShapeDtypeStruct((16, 64), F32),
  )(agg, sk, res, w, b)


def _heads_body(comb_ref, *refs):
  ap = refs[0:14]
  cp = refs[14:28]
  logits_ref = refs[28]
  val_ref = refs[29]
  cb = comb_ref[...]
  kmask = lax.broadcasted_iota(jnp.int32, (16, 16), 1) < U
  inv = 1.0 / math.sqrt(AH // H)

  def head(prm):
    (pW, pb, wq, bq, wk, bk, wv, bv, wo, bo, sW, sb, oW, ob) = prm
    p = jnp.dot(cb, pW[...], preferred_element_type=F32) + pb[0]
    qh = jnp.dot(p, wq[...], preferred_element_type=F32) + bq[0]
    kh = jnp.dot(p, wk[...], preferred_element_type=F32) + bk[0]
    vh = jnp.dot(p, wv[...], preferred_element_type=F32) + bv[0]
    outs = []
    for h in range(H):
      qs = qh[:, h * 8:(h + 1) * 8]
      ks = kh[:, h * 8:(h + 1) * 8]
      vs = vh[:, h * 8:(h + 1) * 8]
      att = lax.dot_general(qs, ks, (((1,), (1,)), ((), ())),
                            preferred_element_type=F32) * inv
      att = jnp.where(kmask, att, -1e30)
      att = att - jnp.max(att, axis=-1, keepdims=True)
      att = jnp.exp(att)
      att = att / jnp.sum(att, axis=-1, keepdims=True)
      outs.append(jnp.dot(att, vs, preferred_element_type=F32))
    o = jnp.concatenate(outs, axis=1)
    o = jnp.dot(o, wo[...], preferred_element_type=F32) + bo[0]
    f = jnp.dot(o, sW[...], preferred_element_type=F32) + sb[0]
    f = jnp.maximum(f, 0.0)
    return jnp.dot(f, oW[...], preferred_element_type=F32) + ob[0]

  logits_ref[...] = head(ap)
  val_ref[...] = head(cp)


def _heads(comb, aprm, cprm):
  args = [comb] + list(aprm) + list(cprm)
  in_specs = []
  for a in args:
    nd = a.ndim
    in_specs.append(pl.BlockSpec(a.shape, (lambda nd_: lambda: (0,) * nd_)(nd)))
  return pl.pallas_call(
      _heads_body,
      in_specs=in_specs,
      out_specs=[
          pl.BlockSpec((16, 1024), lambda: (0, 0)),
          pl.BlockSpec((16, 128), lambda: (0, 0)),
      ],
      out_shape=[
          jax.ShapeDtypeStruct((16, 1024), F32),
          jax.ShapeDtypeStruct((16, 128), F32),
      ],
  )(*args)


_NEG = -1e30


def _edge_body(q_hbm, k_hbm, v_hbm, src_hbm, dst_hbm, rp_hbm, agg_hbm,
               rp_v, dstv, dstv8, srcv, qr, kr, vr,
               aggb, zrow, rowbuf, sem0, sem1, sem2):
  c = lax.axis_index("c")
  s = lax.axis_index("s")
  wid = s * 2 + c
  n0 = wid * NPW
  n1 = n0 + NPW

  pltpu.sync_copy(rp_hbm, rp_v)
  rpv = rp_v[pl.ds(wid, 16)]
  e0 = rpv[0]
  e1 = rpv[1]
  blk0 = (e0 // 8) * 8
  nblk = (e1 - blk0 + (EB - 1)) // EB

  def _zi(i, carry):
    zrow[pl.ds(i * 16, 16)] = jnp.zeros((16,), F32)
    return carry
  lax.fori_loop(0, 32, _zi, 0)

  iota16 = lax.broadcasted_iota(jnp.int32, (16,), 0)
  zero16 = jnp.zeros((16,), F32)
  neg16 = jnp.full((16,), _NEG, F32)

  def _za(i, carry):
    aggb[pl.ds(i * 16, 16)] = jnp.zeros((16,), F32)
    return carry
  lax.fori_loop(0, 32, _za, 0)

  def flush(cur, nxt, d):
    @pl.when(cur >= 0)
    def _():
      rd = 1.0 / (d + 1e-16)
      for hh in range(H):
        rs = rd[hh]
        for jj in range(4):
          o = (hh * 4 + jj) * 16
          rowbuf[pl.ds(o, 16)] = aggb[pl.ds(o, 16)] * rs
          aggb[pl.ds(o, 16)] = jnp.zeros((16,), F32)
      pltpu.sync_copy(rowbuf, agg_hbm.at[cur])

    gap0 = jnp.where(cur >= 0, cur + 1, n0)

    def _gz(nn, carry):
      pltpu.sync_copy(zrow, agg_hbm.at[nn])
      return carry
    lax.fori_loop(gap0, nxt, _gz, 0)

  def estep(e, carry):
    cur, m, d = carry
    dstn = dstv8[pl.ds(e, 16)][0]
    owned = jnp.logical_and(dstn >= n0, dstn < n1)
    newnode = jnp.logical_and(owned, dstn != cur)

    @pl.when(newnode)
    def _():
      flush(cur, dstn, d)

    m = jnp.where(newnode, neg16, m)
    d = jnp.where(newnode, zero16, d)

    a_acc = zero16
    for h in range(H):
      t = qr[e, pl.ds(h * 64, 16)] * kr[e, pl.ds(h * 64, 16)]
      for j2 in range(1, 4):
        t = t + (qr[e, pl.ds(h * 64 + j2 * 16, 16)] *
                 kr[e, pl.ds(h * 64 + j2 * 16, 16)])
      sh = jnp.sum(t) * 0.125
      a_acc = jnp.where(iota16 == h, sh, a_acc)

    mnew = jnp.maximum(m, a_acc)
    f = jnp.exp(m - mnew)
    ex = jnp.exp(a_acc - mnew)
    cur2 = jnp.where(owned, dstn, cur)
    m2 = jnp.where(owned, mnew, m)
    d3 = jnp.where(owned, d * f + ex, d)

    @pl.when(owned)
    def _():
      for h in range(H):
        fh = f[h]
        eh = ex[h]
        for j2 in range(4):
          o = (h * 4 + j2) * 16
          aggb[pl.ds(o, 16)] = (aggb[pl.ds(o, 16)] * fh +
                                eh * vr[e, pl.ds(h * 64 + j2 * 16, 16)])

    return (cur2, m2, d3)

  def pblock(b, carry):
    off = blk0 + b * EB
    pltpu.sync_copy(dst_hbm.at[pl.ds(off, EB)], dstv)
    pltpu.sync_copy(dst_hbm.at[pl.ds(off, 80)], dstv8)
    pltpu.sync_copy(src_hbm.at[pl.ds(off, EB)], srcv)
    cpq = pltpu.async_copy(q_hbm.at[dstv], qr, sem0)
    cpk = pltpu.async_copy(k_hbm.at[srcv], kr, sem1)
    cpv = pltpu.async_copy(v_hbm.at[srcv], vr, sem2)
    cpq.wait()
    cpk.wait()
    cpv.wait()
    return lax.fori_loop(0, EB, estep, carry)

  init = (jnp.int32(-1), neg16, zero16)
  fin = lax.fori_loop(0, nblk, pblock, init)
  flush(fin[0], n1, fin[2])


_edge = functools.partial(
    pl.kernel,
    mesh=plsc.VectorSubcoreMesh(core_axis_name="c", subcore_axis_name="s"),
    compiler_params=pltpu.CompilerParams(
        use_tc_tiling_on_sc=False, needs_layout_passes=False),
    out_type=jax.ShapeDtypeStruct((NPAD, 512), F32),
    scratch_types=[
        pltpu.VMEM((48,), jnp.int32),
        pltpu.VMEM((EB,), jnp.int32),
        pltpu.VMEM((80,), jnp.int32),
        pltpu.VMEM((EB,), jnp.int32),
        pltpu.VMEM((EB, 512), F32),
        pltpu.VMEM((EB, 512), F32),
        pltpu.VMEM((EB, 512), F32),
        pltpu.VMEM((512,), F32),
        pltpu.VMEM((512,), F32),
        pltpu.VMEM((512,), F32),
        pltpu.SemaphoreType.DMA,
        pltpu.SemaphoreType.DMA,
        pltpu.SemaphoreType.DMA,
    ],
)(_edge_body)


def kernel(mission_coords, edge_index, batch, uavs_info, action_mask,
           speeds, dist_matrix, timetogo_matrix, params):
  del batch
  src = edge_index[0]
  dst = edge_index[1]

  order = jnp.argsort(dst)
  dsts = jnp.take(dst, order)
  srcs = jnp.take(src, order)
  dsts_p = jnp.concatenate(
      [dsts, jnp.full((EPAD - E,), NPAD - 1, jnp.int32)])
  srcs_p = jnp.concatenate([srcs, jnp.zeros((EPAD - E,), jnp.int32)])
  bounds = jnp.arange(0, NPAD + 1, NPW, dtype=jnp.int32)
  rp = jnp.searchsorted(dsts, bounds).astype(jnp.int32)
  rp48 = jnp.concatenate([rp, jnp.full((48 - rp.shape[0],), E, jnp.int32)])

  mask_e = action_mask[..., None]
  sp_e = jnp.broadcast_to(speeds[:, None, None], (U, M, 1))
  d_e = dist_matrix[..., None]
  t_e = timetogo_matrix[..., None]
  mc_e = jnp.broadcast_to(mission_coords[None, :, :], (U, M, 2))
  x0 = jnp.concatenate([mc_e, mask_e, sp_e, d_e, t_e], axis=-1).reshape(N, 6)
  x0p = jnp.zeros((NPAD, 128), F32)
  x0p = x0p.at[:N, :6].set(x0)

  w0 = jnp.concatenate(
      [params["l0_Wq"], params["l0_Wk"], params["l0_Wv"],
       params["l0_Wskip"], params["l0_Wres"]], axis=1)
  w0p = jnp.zeros((128, 2560), F32).at[:6, :].set(w0)
  b0 = jnp.concatenate(
      [params["l0_Wq_b"], params["l0_Wk_b"], params["l0_Wv_b"],
       params["l0_Wskip_b"], params["l0_Wres_b"]]).reshape(1, 2560)

  q, k, v, sk, res = _proj0(x0p, w0p, b0)

  w_all = jnp.stack([
      jnp.concatenate(
          [params[f"l{l}_Wq"], params[f"l{l}_Wk"],
           params[f"l{l}_Wv"], params[f"l{l}_Wskip"]], axis=1)
      for l in range(1, NUM_LAYERS)])
  b_all = jnp.stack([
      jnp.concatenate(
          [params[f"l{l}_Wq_b"], params[f"l{l}_Wk_b"],
           params[f"l{l}_Wv_b"], params[f"l{l}_Wskip_b"]]).reshape(1, 2048)
      for l in range(1, NUM_LAYERS)])

  def layer_step(carry, wb):
    q, k, v, sk, res = carry
    wl, bl = wb
    agg = _edge(q, k, v, srcs_p, dsts_p, rp48)
    x, q2, k2, v2, sk2 = _projc(agg, sk, res, wl, bl)
    return (q2, k2, v2, sk2, x), None

  (q, k, v, sk, res), _ = lax.scan(
      layer_step, (q, k, v, sk, res), (w_all, b_all))

  agg = _edge(q, k, v, srcs_p, dsts_p, rp48)
  uav_emb_p = _embred(agg, sk, res, params["Wout"],
                      params["Wout_b"].reshape(1, 64))

  comb = jnp.concatenate(
      [uavs_info, uav_emb_p[:U], speeds[:, None]], axis=1)
  combp = jnp.zeros((16, 128), F32).at[:U, :EMB + 3].set(comb)

  def head_params(pre, out_w, out_b, out_pad):
    ow = jnp.zeros((HID, out_pad), F32).at[:, :out_w.shape[1]].set(out_w)
    ob = jnp.zeros((1, out_pad), F32).at[0, :out_b.shape[0]].set(out_b)
    return (
        jnp.zeros((128, AH), F32).at[:EMB + 3, :].set(params[f"{pre}_proj_W"]),
        params[f"{pre}_proj_b"].reshape(1, AH),
        params[f"{pre}_attn_Wq"], params[f"{pre}_attn_bq"].reshape(1, AH),
        params[f"{pre}_attn_Wk"], params[f"{pre}_attn_bk"].reshape(1, AH),
        params[f"{pre}_attn_Wv"], params[f"{pre}_attn_bv"].reshape(1, AH),
        params[f"{pre}_attn_Wo"], params[f"{pre}_attn_bo"].reshape(1, AH),
        params[f"{pre}_sh_W"], params[f"{pre}_sh_b"].reshape(1, HID),
        ow, ob,
    )

  aprm = head_params("actor", params["actor_out_W"],
                     params["actor_out_b"], 1024)
  cprm = head_params("critic", params["critic_out_W"],
                     params["critic_out_b"], 128)

  logits_p, val_p = _heads(combp, aprm, cprm)
  action_logits = logits_p[:U, :M]
  state_values = val_p[:U, 0]
  return action_logits, state_values
